# baseline (speedup 1.0000x reference)
import jax
import jax.numpy as jnp
from jax.experimental import pallas as pl
from jax.experimental.pallas import tpu as pltpu

Q, NS, D_IN, D_EMB, NWAY = 16384, 6400, 2048, 512, 64
SB = 640
QB = 2048
NSB = NS // SB
NQB = Q // QB


def _fused_kernel(s_ref, t_ref, q_ref, w_ref, out_ref,
                  acc_ref, emb_ref, oh_ref, qp_ref, q2_ref):
    i = pl.program_id(0)
    ph = jax.lax.rem(i, 2)

    @pl.when(i == 0)
    def _init():
        acc_ref[...] = jnp.zeros_like(acc_ref)
        emb_ref[...] = jnp.zeros_like(emb_ref)
        oh_ref[...] = jnp.zeros_like(oh_ref)

    @pl.when(i <= NSB)
    def _support():
        acc_ref[...] += jax.lax.dot_general(
            oh_ref[...], emb_ref[...], (((0,), (0,)), ((), ())),
            preferred_element_type=jnp.float32,
            precision=jax.lax.Precision.HIGHEST)

        t = t_ref[0, 0, :]
        oh = (t[:, None] == jax.lax.broadcasted_iota(jnp.int32, (SB, NWAY), 1))
        oh_ref[...] = oh.astype(jnp.float32)
        emb_ref[...] = jnp.dot(s_ref[...], w_ref[...],
                               preferred_element_type=jnp.float32)

    @pl.when(i == NSB)
    def _finalize():
        sums = acc_ref[...]
        norm = jnp.sqrt(jnp.sum(sums * sums, axis=1, keepdims=True))
        acc_ref[...] = sums / jnp.maximum(norm, 1e-12)

    @pl.when(i > NSB)
    def _query():
        proto = acc_ref[...]

        qp = qp_ref[1 - ph]
        q2 = q2_ref[1 - ph]
        m2 = jnp.sum(proto * proto, axis=1)[None, :]
        d2 = jnp.maximum(q2 + m2 - 2.0 * qp, 1e-12)
        dist = jnp.sqrt(d2)
        dist = dist * dist
        out_ref[...] = jnp.argmin(dist, axis=1).astype(jnp.int32)

        qe = jnp.dot(q_ref[...], w_ref[...],
                     preferred_element_type=jnp.float32)
        qp_ref[ph] = jax.lax.dot_general(
            qe, proto, (((1,), (1,)), ((), ())),
            preferred_element_type=jnp.float32)
        q2_ref[ph] = jnp.sum(qe * qe, axis=1, keepdims=True)


def _clip(x, lo, hi):
    return jnp.minimum(jnp.maximum(x, lo), hi)


def kernel(query_image, support_image, support_target, W, n_way):
    t3 = support_target.astype(jnp.int32).reshape(NSB, 1, SB)

    out = pl.pallas_call(
        _fused_kernel,
        grid=(NSB + NQB + 2,),
        in_specs=[
            pl.BlockSpec((SB, D_IN), lambda i: (jnp.minimum(i, NSB - 1), 0)),
            pl.BlockSpec((1, 1, SB), lambda i: (jnp.minimum(i, NSB - 1), 0, 0)),
            pl.BlockSpec((QB, D_IN),
                         lambda i: (_clip(i - NSB - 1, 0, NQB - 1), 0)),
            pl.BlockSpec((D_IN, D_EMB), lambda i: (0, 0)),
        ],
        out_specs=pl.BlockSpec(
            (QB,), lambda i: (_clip(i - NSB - 2, 0, NQB - 1),)),
        out_shape=jax.ShapeDtypeStruct((Q,), jnp.int32),
        scratch_shapes=[
            pltpu.VMEM((NWAY, D_EMB), jnp.float32),
            pltpu.VMEM((SB, D_EMB), jnp.float32),
            pltpu.VMEM((SB, NWAY), jnp.float32),
            pltpu.VMEM((2, QB, NWAY), jnp.float32),
            pltpu.VMEM((2, QB, 1), jnp.float32),
        ],
        compiler_params=pltpu.CompilerParams(
            dimension_semantics=("arbitrary",)),
    )(support_image, t3, query_image, W)

    return out


# speedup vs baseline: 1.0188x; 1.0188x over previous
import jax
import jax.numpy as jnp
from jax.experimental import pallas as pl
from jax.experimental.pallas import tpu as pltpu

Q, NS, D_IN, D_EMB, NWAY = 16384, 6400, 2048, 512, 64
SB = 640
QB = 2048
NSB = NS // SB
NQB = Q // QB


def _fused_kernel(s_ref, t_ref, q_ref, w_ref, out_ref,
                  acc_ref, qp_ref, q2_ref):
    i = pl.program_id(0)
    ph = jax.lax.rem(i, 2)

    @pl.when(i == 0)
    def _init():
        acc_ref[...] = jnp.zeros_like(acc_ref)

    @pl.when(i < NSB)
    def _support():
        emb = jnp.dot(s_ref[...], w_ref[...],
                      preferred_element_type=jnp.float32)
        t = t_ref[0, 0, :]
        oh = (t[:, None] == jax.lax.broadcasted_iota(jnp.int32, (SB, NWAY), 1))
        oh = oh.astype(jnp.float32)
        acc_ref[...] += jax.lax.dot_general(
            oh, emb, (((0,), (0,)), ((), ())),
            preferred_element_type=jnp.float32,
            precision=jax.lax.Precision.HIGHEST)

    @pl.when(i == NSB - 1)
    def _finalize():
        sums = acc_ref[...]
        norm = jnp.sqrt(jnp.sum(sums * sums, axis=1, keepdims=True))
        acc_ref[...] = sums / jnp.maximum(norm, 1e-12)

    @pl.when(i >= NSB)
    def _query():
        proto = acc_ref[...]

        qp = qp_ref[1 - ph]
        q2 = q2_ref[1 - ph]
        m2 = jnp.sum(proto * proto, axis=1)[None, :]
        d2 = jnp.maximum(q2 + m2 - 2.0 * qp, 1e-12)
        dist = jnp.sqrt(d2)
        dist = dist * dist
        out_ref[...] = jnp.argmin(dist, axis=1).astype(jnp.int32)

        qe = jnp.dot(q_ref[...], w_ref[...],
                     preferred_element_type=jnp.float32)
        qp_ref[ph] = jax.lax.dot_general(
            qe, proto, (((1,), (1,)), ((), ())),
            preferred_element_type=jnp.float32)
        q2_ref[ph] = jnp.sum(qe * qe, axis=1, keepdims=True)


def _clip(x, lo, hi):
    return jnp.minimum(jnp.maximum(x, lo), hi)


def kernel(query_image, support_image, support_target, W, n_way):
    t3 = support_target.astype(jnp.int32).reshape(NSB, 1, SB)

    out = pl.pallas_call(
        _fused_kernel,
        grid=(NSB + NQB + 1,),
        in_specs=[
            pl.BlockSpec((SB, D_IN), lambda i: (jnp.minimum(i, NSB - 1), 0)),
            pl.BlockSpec((1, 1, SB), lambda i: (jnp.minimum(i, NSB - 1), 0, 0)),
            pl.BlockSpec((QB, D_IN),
                         lambda i: (_clip(i - NSB, 0, NQB - 1), 0)),
            pl.BlockSpec((D_IN, D_EMB), lambda i: (0, 0)),
        ],
        out_specs=pl.BlockSpec(
            (QB,), lambda i: (_clip(i - NSB - 1, 0, NQB - 1),)),
        out_shape=jax.ShapeDtypeStruct((Q,), jnp.int32),
        scratch_shapes=[
            pltpu.VMEM((NWAY, D_EMB), jnp.float32),
            pltpu.VMEM((2, QB, NWAY), jnp.float32),
            pltpu.VMEM((2, QB, 1), jnp.float32),
        ],
        compiler_params=pltpu.CompilerParams(
            dimension_semantics=("arbitrary",)),
    )(support_image, t3, query_image, W)

    return out
